# baseline (speedup 1.0000x reference)
import functools

import jax
import jax.numpy as jnp
from jax import lax
from jax.experimental import pallas as pl
from jax.experimental.pallas import tpu as pltpu
from jax.experimental.pallas import tpu_sc as plsc

_SRC_X = 2048
_SRC_Y = 2048
_ROWS = 4096
_COLS = 4096
_N = _ROWS * _COLS

_NC = 2
_NS = 16
_NW = _NC * _NS
_SLAB = _N // _NW

_POS = _SRC_X * _COLS + _SRC_Y
_OWNER = _POS // _SLAB
_CHUNK = _POS - _POS % 16
_LANE = _POS % 16

_mesh = plsc.VectorSubcoreMesh(core_axis_name="c", subcore_axis_name="s")


@functools.partial(
    pl.kernel,
    out_type=jax.ShapeDtypeStruct((_N,), jnp.float32),
    mesh=_mesh,
    scratch_types=[
        pltpu.VMEM((16,), jnp.float32),
        pltpu.VMEM((16,), jnp.float32),
    ],
)
def _scatter_copy(b_hbm, bt_hbm, out_hbm, chunk_v, bt_v):
    wid = lax.axis_index("s") * _NC + lax.axis_index("c")
    base = wid * _SLAB
    pltpu.sync_copy(b_hbm.at[pl.ds(base, _SLAB)], out_hbm.at[pl.ds(base, _SLAB)])

    @pl.when(wid == _OWNER)
    def _():
        pltpu.sync_copy(b_hbm.at[pl.ds(_CHUNK, 16)], chunk_v)
        pltpu.sync_copy(bt_hbm, bt_v)
        lanes = lax.broadcasted_iota(jnp.int32, (16,), 0)
        chunk_v[...] = jnp.where(lanes == _LANE, bt_v[...], chunk_v[...])
        pltpu.sync_copy(chunk_v, out_hbm.at[pl.ds(_CHUNK, 16)])


def kernel(B, Bt):
    b_flat = B.reshape(_N)
    bt_vec = jnp.broadcast_to(Bt.reshape(()), (16,))
    out = _scatter_copy(b_flat, bt_vec)
    return out.reshape(B.shape)


# speedup vs baseline: 51.7097x; 51.7097x over previous
import functools

import jax
import jax.numpy as jnp
from jax import lax
from jax.experimental import pallas as pl
from jax.experimental.pallas import tpu as pltpu

_SRC_X = 2048
_SRC_Y = 2048
_ROWS = 4096
_COLS = 4096

_R = 512
_G = _ROWS // _R
_TBLK = _SRC_X // _R
_LR = _SRC_X % _R
_LR8 = (_LR // 8) * 8


def _body(bt_ref, b_ref, o_ref):
    o_ref[...] = b_ref[...]

    @pl.when(pl.program_id(0) == _TBLK)
    def _():
        sub = b_ref[pl.ds(_LR8, 8), pl.ds(_SRC_Y, 128)]
        ri = lax.broadcasted_iota(jnp.int32, (8, 128), 0)
        ci = lax.broadcasted_iota(jnp.int32, (8, 128), 1)
        hit = (ri == _LR - _LR8) & (ci == 0)
        o_ref[pl.ds(_LR8, 8), pl.ds(_SRC_Y, 128)] = jnp.where(
            hit, bt_ref[0, 0], sub)


@jax.jit
def _scatter_copy(bt, b2d):
    return pl.pallas_call(
        _body,
        grid=(_G,),
        in_specs=[
            pl.BlockSpec(memory_space=pltpu.SMEM),
            pl.BlockSpec((_R, _COLS), lambda i: (i, 0)),
        ],
        out_specs=pl.BlockSpec((_R, _COLS), lambda i: (i, 0)),
        out_shape=jax.ShapeDtypeStruct((_ROWS, _COLS), jnp.float32),
    )(bt, b2d)


def kernel(B, Bt):
    out = _scatter_copy(Bt, B.reshape(_ROWS, _COLS))
    return out.reshape(B.shape)
